# baseline (speedup 1.0000x reference)
import functools

import jax
import jax.numpy as jnp
from jax import lax
from jax.experimental import pallas as pl
from jax.experimental.pallas import tpu as pltpu
from jax.experimental.pallas import tpu_sc as plsc

_EMBED = 64
_CHUNK = 128


@functools.lru_cache(maxsize=None)
def _make_gather(n_workers: int, n_chunks: int, vocab: int):
    b_per_w = n_chunks * _CHUNK
    total = n_workers * b_per_w
    mesh = plsc.VectorSubcoreMesh(core_axis_name="c", subcore_axis_name="s")

    @functools.partial(
        pl.kernel,
        mesh=mesh,
        out_type=jax.ShapeDtypeStruct((total, _EMBED), jnp.float32),
        scratch_types=[
            pltpu.VMEM((n_chunks, _CHUNK), jnp.int32),
            pltpu.VMEM((_CHUNK, _EMBED), jnp.float32),
            pltpu.VMEM((_CHUNK, _EMBED), jnp.float32),
            pltpu.SemaphoreType.DMA,
            pltpu.SemaphoreType.DMA,
        ],
        compiler_params=pltpu.CompilerParams(use_tc_tiling_on_sc=False),
    )
    def gather(idx_hbm, table_hbm, out_hbm, idx_v, rows0, rows1, sem0, sem1):
        n_cores = 2
        wid = lax.axis_index("s") * n_cores + lax.axis_index("c")
        base = wid * b_per_w
        pltpu.sync_copy(idx_hbm.at[wid], idx_v)

        def body(jj, carry):
            j0 = jj * 2
            j1 = j0 + 1
            cp0 = pltpu.async_copy(table_hbm.at[idx_v.at[j0]], rows0, sem0)
            cp1 = pltpu.async_copy(table_hbm.at[idx_v.at[j1]], rows1, sem1)
            cp0.wait()
            pltpu.sync_copy(rows0, out_hbm.at[pl.ds(base + j0 * _CHUNK, _CHUNK)])
            cp1.wait()
            pltpu.sync_copy(rows1, out_hbm.at[pl.ds(base + j1 * _CHUNK, _CHUNK)])
            return carry

        lax.fori_loop(0, n_chunks // 2, body, 0)
        if n_chunks % 2:
            j0 = n_chunks - 1
            pltpu.async_copy(table_hbm.at[idx_v.at[j0]], rows0, sem0).wait()
            pltpu.sync_copy(rows0, out_hbm.at[pl.ds(base + j0 * _CHUNK, _CHUNK)])

    return gather


def kernel(x, table):
    batch, seq = x.shape
    vocab, embed = table.shape
    assert embed == _EMBED
    total = batch * seq
    n_workers = 32
    assert total % (n_workers * _CHUNK) == 0
    n_chunks = total // (n_workers * _CHUNK)
    idx = x.reshape(n_workers, n_chunks, _CHUNK).astype(jnp.int32)
    out = _make_gather(n_workers, n_chunks, vocab)(idx, table)
    return out.reshape(batch, seq, embed)
